# Initial kernel scaffold; baseline (speedup 1.0000x reference)
import jax
import jax.numpy as jnp
from jax.experimental import pallas as pl


def kernel(dist, angle, idx_t, index_t):
    raise NotImplementedError("write your pallas kernel here")


# speedup vs baseline: 13.3372x; 13.3372x over previous
import functools

import jax
import jax.numpy as jnp
from jax import lax
from jax.experimental import pallas as pl
from jax.experimental.pallas import tpu as pltpu
from jax.experimental.pallas import tpu_sc as plsc

H = 16384
W = 200
K = 21
C = K * W + 2 * W
L = 16
NC, NS = 2, 16
NW = NC * NS
ROWS_PER_W = H // NW
RB = 8
NBLK = ROWS_PER_W // RB
NG = (W + L - 1) // L
IN_PAD = RB * W + 8
OUT_WORDS = RB * C


def _sc_body(dist_hbm, angle_hbm, idx_hbm, ind_hbm, out_hbm,
             dist_v, angle_v, idx_v, ind_v, out_v):
    wid = lax.axis_index("s") * NC + lax.axis_index("c")

    lane = lax.iota(jnp.int32, L)
    lane21 = lane * 21
    tail_mask = lane < jnp.int32(W - (NG - 1) * L)
    ones = jnp.full((L,), 1.0, jnp.float32)
    zeros = jnp.zeros((L,), jnp.float32)

    def zero_body(i, c):
        out_v[pl.ds(i * L, L)] = zeros
        return c
    lax.fori_loop(0, OUT_WORDS // L, zero_body, 0)

    def onehot_pos(r_out, r_in, g):
        idxv = idx_v[pl.ds(r_in + g * L, L)]
        return r_out + g * L * K + lane21 + idxv

    def block_body(b, carry):
        base = wid * ROWS_PER_W + b * RB
        in_off = base * W
        out_off = base * C
        pltpu.sync_copy(dist_hbm.at[pl.ds(in_off, RB * W)],
                        dist_v.at[pl.ds(0, RB * W)])
        pltpu.sync_copy(angle_hbm.at[pl.ds(in_off, RB * W)],
                        angle_v.at[pl.ds(0, RB * W)])
        pltpu.sync_copy(idx_hbm.at[pl.ds(in_off, RB * W)],
                        idx_v.at[pl.ds(0, RB * W)])
        pltpu.sync_copy(ind_hbm.at[pl.ds(in_off, RB * W)],
                        ind_v.at[pl.ds(0, RB * W)])

        def row_body(r, c):
            r_in = r * W
            r_out = r * C
            for g in range(NG):
                off = g * L
                mk = None if g < NG - 1 else tail_mask
                pos = onehot_pos(r_out, r_in, g)
                plsc.store_scatter(out_v, [pos], ones, mask=mk)
                indv = ind_v[pl.ds(r_in + off, L)]
                gsrc = r_in + indv
                dd = plsc.load_gather(dist_v, [gsrc], mask=mk) * jnp.float32(0.1)
                aa = plsc.load_gather(angle_v, [gsrc], mask=mk) * jnp.float32(1.0 / 3.0)
                if mk is None:
                    out_v[pl.ds(r_out + K * W + off, L)] = dd
                    out_v[pl.ds(r_out + K * W + W + off, L)] = aa
                else:
                    plsc.store_scatter(out_v, [r_out + K * W + off + lane], dd, mask=mk)
                    plsc.store_scatter(out_v, [r_out + K * W + W + off + lane], aa, mask=mk)
            return c
        lax.fori_loop(0, RB, row_body, 0)

        pltpu.sync_copy(out_v, out_hbm.at[pl.ds(out_off, OUT_WORDS)])

        def rz_body(r, c):
            r_in = r * W
            r_out = r * C
            for g in range(NG):
                mk = None if g < NG - 1 else tail_mask
                pos = onehot_pos(r_out, r_in, g)
                plsc.store_scatter(out_v, [pos], zeros, mask=mk)
            return c
        lax.fori_loop(0, RB, rz_body, 0)
        return carry

    lax.fori_loop(0, NBLK, block_body, 0)


_sc_call = pl.kernel(
    _sc_body,
    out_type=jax.ShapeDtypeStruct((H * C,), jnp.float32),
    mesh=plsc.VectorSubcoreMesh(core_axis_name="c", subcore_axis_name="s",
                                num_cores=NC, num_subcores=NS),
    scratch_types=[
        pltpu.VMEM((IN_PAD,), jnp.float32),
        pltpu.VMEM((IN_PAD,), jnp.float32),
        pltpu.VMEM((IN_PAD,), jnp.int32),
        pltpu.VMEM((IN_PAD,), jnp.int32),
        pltpu.VMEM((OUT_WORDS,), jnp.float32),
    ],
    compiler_params=pltpu.CompilerParams(needs_layout_passes=False),
)


@jax.jit
def kernel(dist, angle, idx_t, index_t):
    flat = _sc_call(dist.reshape(-1),
                    angle.reshape(-1),
                    idx_t.astype(jnp.int32).reshape(-1),
                    index_t.astype(jnp.int32).reshape(-1))
    return flat.reshape(H, C)


# speedup vs baseline: 17.6676x; 1.3247x over previous
import jax
import jax.numpy as jnp
from jax import lax
from jax.experimental import pallas as pl
from jax.experimental.pallas import tpu as pltpu
from jax.experimental.pallas import tpu_sc as plsc

H = 16384
W = 200
K = 21
C = K * W + 2 * W
L = 16
NC, NS = 2, 16
NW = NC * NS
ROWS_PER_W = H // NW
RB = 8
NBLK = ROWS_PER_W // RB
NG = (W + L - 1) // L
IN_PAD = RB * W + 8
OUT_WORDS = RB * C
POS_W = NG * L


def _sc_body(dist_hbm, angle_hbm, idx_hbm, ind_hbm, out_hbm,
             din0, ain0, iin0, nin0, din1, ain1, iin1, nin1,
             out0, out1, pos0, pos1,
             sem_in0, sem_in1, sem_out0, sem_out1):
    wid = lax.axis_index("s") * NC + lax.axis_index("c")

    lane = lax.iota(jnp.int32, L)
    lane21 = lane * 21
    tail_mask = lane < jnp.int32(W - (NG - 1) * L)
    ones = jnp.full((L,), 1.0, jnp.float32)
    zeros = jnp.zeros((L,), jnp.float32)

    bufs = ((din0, ain0, iin0, nin0, out0, pos0, sem_in0, sem_out0),
            (din1, ain1, iin1, nin1, out1, pos1, sem_in1, sem_out1))

    def zero_body(i, c):
        out0[pl.ds(i * L, L)] = zeros
        out1[pl.ds(i * L, L)] = zeros
        return c
    lax.fori_loop(0, OUT_WORDS // L, zero_body, 0)

    def start_in(b, B):
        in_off = (wid * ROWS_PER_W + b * RB) * W
        for hbm, v in zip((dist_hbm, angle_hbm, idx_hbm, ind_hbm), B[0:4]):
            pltpu.async_copy(hbm.at[pl.ds(in_off, RB * W)],
                             v.at[pl.ds(0, RB * W)], B[6])

    def wait_in(B):
        for hbm, v in zip((dist_hbm, angle_hbm, idx_hbm, ind_hbm), B[0:4]):
            pltpu.make_async_copy(hbm.at[pl.ds(0, RB * W)],
                                  v.at[pl.ds(0, RB * W)], B[6]).wait()

    def start_out(b, B):
        out_off = (wid * ROWS_PER_W + b * RB) * C
        pltpu.async_copy(B[4], out_hbm.at[pl.ds(out_off, OUT_WORDS)], B[7])

    def wait_out(B):
        pltpu.make_async_copy(B[4], out_hbm.at[pl.ds(0, OUT_WORDS)], B[7]).wait()

    def compute_block(B):
        d_v, a_v, i_v, n_v, o_v, p_v = B[0:6]

        def row_body(r, c):
            r_in = r * W
            r_out = r * C
            r_pos = r * POS_W
            for g in range(NG):
                off = g * L
                mk = None if g < NG - 1 else tail_mask
                idxv = i_v[pl.ds(r_in + off, L)]
                pos = r_out + g * L * K + lane21 + idxv
                p_v[pl.ds(r_pos + off, L)] = pos
                plsc.store_scatter(o_v, [pos], ones, mask=mk)
                indv = n_v[pl.ds(r_in + off, L)]
                gsrc = r_in + indv
                dd = plsc.load_gather(d_v, [gsrc], mask=mk) * jnp.float32(0.1)
                aa = plsc.load_gather(a_v, [gsrc], mask=mk) * jnp.float32(1.0 / 3.0)
                if mk is None:
                    o_v[pl.ds(r_out + K * W + off, L)] = dd
                    o_v[pl.ds(r_out + K * W + W + off, L)] = aa
                else:
                    plsc.store_scatter(o_v, [r_out + K * W + off + lane], dd, mask=mk)
                    plsc.store_scatter(o_v, [r_out + K * W + W + off + lane], aa, mask=mk)
            return c
        lax.fori_loop(0, RB, row_body, 0)

    def rezero_block(B):
        o_v, p_v = B[4], B[5]

        def rz_body(r, c):
            r_pos = r * POS_W
            for g in range(NG):
                mk = None if g < NG - 1 else tail_mask
                pos = p_v[pl.ds(r_pos + g * L, L)]
                plsc.store_scatter(o_v, [pos], zeros, mask=mk)
            return c
        lax.fori_loop(0, RB, rz_body, 0)

    start_in(0, bufs[0])
    start_in(1, bufs[1])

    def outer(o, carry):
        for phase in range(2):
            b = o * 2 + phase
            B = bufs[phase]
            wait_in(B)

            @pl.when(b >= 2)
            def _():
                wait_out(B)
                rezero_block(B)

            compute_block(B)
            start_out(b, B)

            @pl.when(b + 2 < NBLK)
            def _():
                start_in(b + 2, B)
        return carry

    lax.fori_loop(0, NBLK // 2, outer, 0)
    wait_out(bufs[0])
    wait_out(bufs[1])


_sc_call = pl.kernel(
    _sc_body,
    out_type=jax.ShapeDtypeStruct((H * C,), jnp.float32),
    mesh=plsc.VectorSubcoreMesh(core_axis_name="c", subcore_axis_name="s",
                                num_cores=NC, num_subcores=NS),
    scratch_types=[
        pltpu.VMEM((IN_PAD,), jnp.float32),
        pltpu.VMEM((IN_PAD,), jnp.float32),
        pltpu.VMEM((IN_PAD,), jnp.int32),
        pltpu.VMEM((IN_PAD,), jnp.int32),
        pltpu.VMEM((IN_PAD,), jnp.float32),
        pltpu.VMEM((IN_PAD,), jnp.float32),
        pltpu.VMEM((IN_PAD,), jnp.int32),
        pltpu.VMEM((IN_PAD,), jnp.int32),
        pltpu.VMEM((OUT_WORDS,), jnp.float32),
        pltpu.VMEM((OUT_WORDS,), jnp.float32),
        pltpu.VMEM((RB * POS_W,), jnp.int32),
        pltpu.VMEM((RB * POS_W,), jnp.int32),
        pltpu.SemaphoreType.DMA,
        pltpu.SemaphoreType.DMA,
        pltpu.SemaphoreType.DMA,
        pltpu.SemaphoreType.DMA,
    ],
    compiler_params=pltpu.CompilerParams(needs_layout_passes=False),
)


@jax.jit
def kernel(dist, angle, idx_t, index_t):
    flat = _sc_call(dist.reshape(-1),
                    angle.reshape(-1),
                    idx_t.astype(jnp.int32).reshape(-1),
                    index_t.astype(jnp.int32).reshape(-1))
    return flat.reshape(H, C)


# speedup vs baseline: 25.0027x; 1.4152x over previous
import jax
import jax.numpy as jnp
from jax import lax
from jax.experimental import pallas as pl
from jax.experimental.pallas import tpu as pltpu
from jax.experimental.pallas import tpu_sc as plsc

H = 16384
W = 200
K = 21
C = K * W + 2 * W
L = 16
NC, NS = 2, 16
NW = NC * NS
ROWS_PER_W = H // NW
RB = 8
NBLK = ROWS_PER_W // RB
NG = (W + L - 1) // L
POS_W = NG * L
CG = (C + L - 1) // L


def _sc_body(dist_hbm, angle_hbm, idx_hbm, ind_hbm, out_hbm,
             din0, ain0, iin0, nin0, din1, ain1, iin1, nin1,
             out0, out1, pos0, pos1,
             sem_in0, sem_in1, sem_out0, sem_out1):
    wid = lax.axis_index("s") * NC + lax.axis_index("c")

    lane = lax.iota(jnp.int32, L)
    lane21 = lane * 21
    tail_mask = lane < jnp.int32(W - (NG - 1) * L)
    ctail_mask = lane < jnp.int32(C - (CG - 1) * L)
    ones = jnp.full((L,), 1.0, jnp.float32)
    zeros = jnp.zeros((L,), jnp.float32)

    bufs = ((din0, ain0, iin0, nin0, out0, pos0, sem_in0, sem_out0),
            (din1, ain1, iin1, nin1, out1, pos1, sem_in1, sem_out1))

    def zero_row(r, c):
        rvec = jnp.full((L,), 0, jnp.int32) + r
        for ov in (out0, out1):
            def zero_grp(g, cc):
                cvec = g * L + lane
                plsc.store_scatter(ov, [rvec, cvec], zeros, mask=cvec < C)
                return cc
            lax.fori_loop(0, CG, zero_grp, 0)
        return c
    lax.fori_loop(0, RB, zero_row, 0)

    ins = (dist_hbm, angle_hbm, idx_hbm, ind_hbm)

    def start_in(b, B):
        base = wid * ROWS_PER_W + b * RB
        for hbm, v in zip(ins, B[0:4]):
            pltpu.async_copy(hbm.at[pl.ds(base, RB)], v, B[6])

    def wait_in(B):
        for hbm, v in zip(ins, B[0:4]):
            pltpu.make_async_copy(hbm.at[pl.ds(0, RB)], v, B[6]).wait()

    def start_out(b, B):
        base = wid * ROWS_PER_W + b * RB
        pltpu.async_copy(B[4], out_hbm.at[pl.ds(base, RB)], B[7])

    def wait_out(B):
        pltpu.make_async_copy(B[4], out_hbm.at[pl.ds(0, RB)], B[7]).wait()

    def compute_block(B):
        d_v, a_v, i_v, n_v, o_v, p_v = B[0:6]

        def row_body(r, c):
            rvec = jnp.full((L,), 0, jnp.int32) + r
            r_pos = r * POS_W
            for g in range(NG):
                off = g * L
                mk = None if g < NG - 1 else tail_mask
                jvec = off + lane
                idxv = plsc.load_gather(i_v, [rvec, jvec], mask=mk)
                cpos = g * L * K + lane21 + idxv
                p_v[pl.ds(r_pos + off, L)] = cpos
                plsc.store_scatter(o_v, [rvec, cpos], ones, mask=mk)
                indv = plsc.load_gather(n_v, [rvec, jvec], mask=mk)
                dd = plsc.load_gather(d_v, [rvec, indv], mask=mk) * jnp.float32(0.1)
                aa = plsc.load_gather(a_v, [rvec, indv], mask=mk) * jnp.float32(1.0 / 3.0)
                plsc.store_scatter(o_v, [rvec, K * W + jvec], dd, mask=mk)
                plsc.store_scatter(o_v, [rvec, K * W + W + jvec], aa, mask=mk)
            return c
        lax.fori_loop(0, RB, row_body, 0)

    def rezero_block(B):
        o_v, p_v = B[4], B[5]

        def rz_body(r, c):
            rvec = jnp.full((L,), 0, jnp.int32) + r
            r_pos = r * POS_W
            for g in range(NG):
                mk = None if g < NG - 1 else tail_mask
                cpos = p_v[pl.ds(r_pos + g * L, L)]
                plsc.store_scatter(o_v, [rvec, cpos], zeros, mask=mk)
            return c
        lax.fori_loop(0, RB, rz_body, 0)

    start_in(0, bufs[0])
    start_in(1, bufs[1])

    def outer(o, carry):
        for phase in range(2):
            b = o * 2 + phase
            B = bufs[phase]
            wait_in(B)

            @pl.when(b >= 2)
            def _():
                wait_out(B)
                rezero_block(B)

            compute_block(B)
            start_out(b, B)

            @pl.when(b + 2 < NBLK)
            def _():
                start_in(b + 2, B)
        return carry

    lax.fori_loop(0, NBLK // 2, outer, 0)
    wait_out(bufs[0])
    wait_out(bufs[1])


_sc_call = pl.kernel(
    _sc_body,
    out_type=jax.ShapeDtypeStruct((H, C), jnp.float32),
    mesh=plsc.VectorSubcoreMesh(core_axis_name="c", subcore_axis_name="s",
                                num_cores=NC, num_subcores=NS),
    scratch_types=[
        pltpu.VMEM((RB, W), jnp.float32),
        pltpu.VMEM((RB, W), jnp.float32),
        pltpu.VMEM((RB, W), jnp.int32),
        pltpu.VMEM((RB, W), jnp.int32),
        pltpu.VMEM((RB, W), jnp.float32),
        pltpu.VMEM((RB, W), jnp.float32),
        pltpu.VMEM((RB, W), jnp.int32),
        pltpu.VMEM((RB, W), jnp.int32),
        pltpu.VMEM((RB, C), jnp.float32),
        pltpu.VMEM((RB, C), jnp.float32),
        pltpu.VMEM((RB * POS_W,), jnp.int32),
        pltpu.VMEM((RB * POS_W,), jnp.int32),
        pltpu.SemaphoreType.DMA,
        pltpu.SemaphoreType.DMA,
        pltpu.SemaphoreType.DMA,
        pltpu.SemaphoreType.DMA,
    ],
    compiler_params=pltpu.CompilerParams(needs_layout_passes=False),
)


@jax.jit
def kernel(dist, angle, idx_t, index_t):
    return _sc_call(dist, angle,
                    idx_t.astype(jnp.int32),
                    index_t.astype(jnp.int32))


# speedup vs baseline: 67.8977x; 2.7156x over previous
import jax
import jax.numpy as jnp
from jax import lax
from jax.experimental import pallas as pl
from jax.experimental.pallas import tpu as pltpu
from jax.experimental.pallas import tpu_sc as plsc

H = 16384
W = 200
K = 21
C = K * W + 2 * W
L = 16
NC, NS = 2, 16
NW = NC * NS
COLS_PER_W = H // NW
IB = 128
LG = IB // L
NBLK = COLS_PER_W // IB
CH = 8
NCH = W // CH
OH = K * CH
SR = OH + 2 * CH


def _sc_body(dist_hbm, angle_hbm, idx_hbm, ind_hbm, out_hbm,
             d_v, a_v, i_c0, n_c0, i_c1, n_c1,
             out0, out1, pos0, pos1,
             sem_da, sem_ic0, sem_ic1, sem_out0, sem_out1):
    wid = lax.axis_index("s") * NC + lax.axis_index("c")
    i_base_w = wid * COLS_PER_W

    lane = lax.iota(jnp.int32, L)
    glane = [g * L + lane for g in range(LG)]
    ones = jnp.full((L,), 1.0, jnp.float32)
    zeros = jnp.zeros((L,), jnp.float32)

    outb = ((out0, pos0, sem_out0), (out1, pos1, sem_out1))
    inb = ((i_c0, n_c0, sem_ic0), (i_c1, n_c1, sem_ic1))

    def zero_body(r, c):
        for g in range(LG):
            out0[r, pl.ds(g * L, L)] = zeros
            out1[r, pl.ds(g * L, L)] = zeros
        return c
    lax.fori_loop(0, SR, zero_body, 0)

    def start_da(k):
        ib = i_base_w + k * IB
        pltpu.async_copy(dist_hbm.at[pl.ds(0, W), pl.ds(ib, IB)], d_v, sem_da)
        pltpu.async_copy(angle_hbm.at[pl.ds(0, W), pl.ds(ib, IB)], a_v, sem_da)

    def wait_da():
        pltpu.make_async_copy(dist_hbm.at[pl.ds(0, W), pl.ds(0, IB)], d_v,
                              sem_da).wait()
        pltpu.make_async_copy(angle_hbm.at[pl.ds(0, W), pl.ds(0, IB)], a_v,
                              sem_da).wait()

    def start_ic(k, c, Bi):
        ib = i_base_w + k * IB
        pltpu.async_copy(idx_hbm.at[pl.ds(c * CH, CH), pl.ds(ib, IB)],
                         Bi[0], Bi[2])
        pltpu.async_copy(ind_hbm.at[pl.ds(c * CH, CH), pl.ds(ib, IB)],
                         Bi[1], Bi[2])

    def wait_ic(Bi):
        pltpu.make_async_copy(idx_hbm.at[pl.ds(0, CH), pl.ds(0, IB)],
                              Bi[0], Bi[2]).wait()
        pltpu.make_async_copy(ind_hbm.at[pl.ds(0, CH), pl.ds(0, IB)],
                              Bi[1], Bi[2]).wait()

    def out_parts(k, c, Bo):
        ib = i_base_w + k * IB
        o_v = Bo[0]
        return (
            (o_v.at[pl.ds(0, OH), pl.ds(0, IB)],
             out_hbm.at[pl.ds(c * OH, OH), pl.ds(ib, IB)]),
            (o_v.at[pl.ds(OH, CH), pl.ds(0, IB)],
             out_hbm.at[pl.ds(K * W + c * CH, CH), pl.ds(ib, IB)]),
            (o_v.at[pl.ds(OH + CH, CH), pl.ds(0, IB)],
             out_hbm.at[pl.ds(K * W + W + c * CH, CH), pl.ds(ib, IB)]),
        )

    def start_out(k, c, Bo):
        for src, dst in out_parts(k, c, Bo):
            pltpu.async_copy(src, dst, Bo[2])

    def wait_out(Bo):
        for src, dst in out_parts(0, 0, Bo):
            pltpu.make_async_copy(src, dst, Bo[2]).wait()

    def compute_pass(Bo, Bi):
        o_v, p_v = Bo[0:2]
        i_c, n_c = Bi[0:2]
        for jj in range(CH):
            for g in range(LG):
                gl = glane[g]
                idxv = i_c[jj, pl.ds(g * L, L)]
                indv = n_c[jj, pl.ds(g * L, L)]
                cv = jj * K + idxv
                p_v[pl.ds((jj * LG + g) * L, L)] = cv
                plsc.store_scatter(o_v, [cv, gl], ones)
                dd = plsc.load_gather(d_v, [indv, gl]) * jnp.float32(0.1)
                aa = plsc.load_gather(a_v, [indv, gl]) * jnp.float32(1.0 / 3.0)
                o_v[OH + jj, pl.ds(g * L, L)] = dd
                o_v[OH + CH + jj, pl.ds(g * L, L)] = aa

    def rezero_pass(Bo):
        o_v, p_v = Bo[0:2]
        for jj in range(CH):
            for g in range(LG):
                cv = p_v[pl.ds((jj * LG + g) * L, L)]
                plsc.store_scatter(o_v, [cv, glane[g]], zeros)

    def blk(k, carry):
        start_da(k)
        start_ic(k, 0, inb[0])
        start_ic(k, 1, inb[1])
        wait_da()

        def chunk(c, cc):
            p = k * NCH + c
            q = lax.rem(c, 2)
            for P in range(2):
                @pl.when(q == P)
                def _():
                    Bo = outb[P]
                    Bi = inb[P]
                    wait_ic(Bi)

                    @pl.when(p >= 2)
                    def _():
                        wait_out(Bo)
                        rezero_pass(Bo)

                    compute_pass(Bo, Bi)
                    start_out(k, c, Bo)

                    @pl.when(c + 2 < NCH)
                    def _():
                        start_ic(k, c + 2, Bi)
            return cc
        lax.fori_loop(0, NCH, chunk, 0)
        return carry

    lax.fori_loop(0, NBLK, blk, 0)
    wait_out(outb[0])
    wait_out(outb[1])


_sc_call = pl.kernel(
    _sc_body,
    out_type=jax.ShapeDtypeStruct((C, H), jnp.float32),
    mesh=plsc.VectorSubcoreMesh(core_axis_name="c", subcore_axis_name="s",
                                num_cores=NC, num_subcores=NS),
    scratch_types=[
        pltpu.VMEM((W, IB), jnp.float32),
        pltpu.VMEM((W, IB), jnp.float32),
        pltpu.VMEM((CH, IB), jnp.int32),
        pltpu.VMEM((CH, IB), jnp.int32),
        pltpu.VMEM((CH, IB), jnp.int32),
        pltpu.VMEM((CH, IB), jnp.int32),
        pltpu.VMEM((SR, IB), jnp.float32),
        pltpu.VMEM((SR, IB), jnp.float32),
        pltpu.VMEM((CH * LG * L,), jnp.int32),
        pltpu.VMEM((CH * LG * L,), jnp.int32),
        pltpu.SemaphoreType.DMA,
        pltpu.SemaphoreType.DMA,
        pltpu.SemaphoreType.DMA,
        pltpu.SemaphoreType.DMA,
        pltpu.SemaphoreType.DMA,
    ],
    compiler_params=pltpu.CompilerParams(needs_layout_passes=False),
)


@jax.jit
def kernel(dist, angle, idx_t, index_t):
    out_t = _sc_call(dist.T, angle.T,
                     idx_t.astype(jnp.int32).T,
                     index_t.astype(jnp.int32).T)
    return out_t.T
